# baseline (speedup 1.0000x reference)
import functools

import jax
import jax.numpy as jnp
from jax import lax
from jax.experimental import pallas as pl
from jax.experimental.pallas import tpu as pltpu
from jax.experimental.pallas import tpu_sc as plsc

B = 16384
F = 64
NC = 2
NS = 16
NW = NC * NS
BPW = B // NW
WIN = 48


@functools.lru_cache(maxsize=2)
def _make_gather(n_tables):
    mesh = plsc.VectorSubcoreMesh(core_axis_name="c", subcore_axis_name="s")

    @functools.partial(
        pl.kernel,
        mesh=mesh,
        out_type=[jax.ShapeDtypeStruct((B, F), jnp.float32)] * n_tables,
        scratch_types=[
            pltpu.VMEM((BPW + 16,), jnp.int32),
            pltpu.VMEM((BPW, F), jnp.float32),
            pltpu.SemaphoreType.DMA,
        ],
    )
    def _gather(*args):
        tables = args[:n_tables]
        idxs = args[n_tables:2 * n_tables]
        outs = args[2 * n_tables:3 * n_tables]
        iv, rows, sem = args[3 * n_tables:]
        wid = lax.axis_index("s") * NC + lax.axis_index("c")
        base = wid * BPW

        def one_table(table, idx, out):
            pltpu.sync_copy(idx.at[pl.ds(base, BPW)], iv.at[pl.ds(0, BPW)])

            def step(r, _):
                s = iv[pl.ds(r, 16)][0]
                pltpu.async_copy(
                    table.at[pl.ds(s, 1)], rows.at[pl.ds(r, 1)], sem)

                @pl.when(r >= WIN)
                def _():
                    pltpu.make_async_copy(
                        table.at[pl.ds(0, 1)], rows.at[pl.ds(0, 1)],
                        sem).wait()

                return 0

            lax.fori_loop(0, BPW, step, 0)
            for _ in range(WIN):
                pltpu.make_async_copy(
                    table.at[pl.ds(0, 1)], rows.at[pl.ds(0, 1)], sem).wait()
            pltpu.sync_copy(rows, out.at[pl.ds(base, BPW)])

        for t, i, o in zip(tables, idxs, outs):
            one_table(t, i, o)

    return _gather


def _transpose_body(t_ref, o_ref):
    eye = jnp.eye(F, dtype=jnp.float32)
    o_ref[...] = jax.lax.dot_general(
        t_ref[...], eye, (((0,), (0,)), ((), ())),
        preferred_element_type=jnp.float32)


def _transpose(tT, blk):
    n = tT.shape[1]
    grid = (n + blk - 1) // blk
    return pl.pallas_call(
        _transpose_body,
        grid=(grid,),
        in_specs=[pl.BlockSpec((F, blk), lambda i: (0, i))],
        out_specs=pl.BlockSpec((blk, F), lambda i: (i, 0)),
        out_shape=jax.ShapeDtypeStruct((n, F), jnp.float32),
    )(tT)


BT = 2048
GRID = B // BT


def _mlp_body(ue, me, ge, w1u, w1m, w1g, b1, w2, b2, w3, b3, out):
    x = (jnp.dot(ue[...], w1u[...], preferred_element_type=jnp.float32)
         + jnp.dot(me[...], w1m[...], preferred_element_type=jnp.float32)
         + jnp.dot(ge[...], w1g[...], preferred_element_type=jnp.float32)
         + b1[...])
    x = jnp.maximum(x, 0.0)
    x = jnp.maximum(
        jnp.dot(x, w2[...], preferred_element_type=jnp.float32) + b2[...], 0.0)
    out[...] = jnp.dot(x, w3[...], preferred_element_type=jnp.float32) + b3[...]


def _mlp(ue, me, ge, w1u, w1m, w1g, b1, w2, b2, w3, b3, *, interpret=False):
    full = lambda shape: pl.BlockSpec(shape, lambda i: (0, 0))
    return pl.pallas_call(
        _mlp_body,
        grid=(GRID,),
        in_specs=[
            pl.BlockSpec((BT, F), lambda i: (i, 0)),
            pl.BlockSpec((BT, F), lambda i: (i, 0)),
            pl.BlockSpec((BT, F), lambda i: (i, 0)),
            full((F, 128)),
            full((F, 128)),
            full((F, 128)),
            full((1, 128)),
            full((128, F)),
            full((1, F)),
            full((F, 1)),
            full((1, 1)),
        ],
        out_specs=pl.BlockSpec((BT, 1), lambda i: (i, 0)),
        out_shape=jax.ShapeDtypeStruct((B, 1), jnp.float32),
        interpret=interpret,
    )(ue, me, ge, w1u, w1m, w1g, b1, w2, b2, w3, b3)


def kernel(user, movie, genres, user_table, movie_table, genre_table,
           W1, b1, W2, b2, W3, b3):
    mt_rm = _transpose(movie_table.T, 16384)
    gt_rm = _transpose(genre_table.T, 1024)
    ut_rm = _transpose(user_table.T, 32768)
    me, ge = _make_gather(2)(mt_rm, gt_rm, movie, genres)
    ue, = _make_gather(1)(ut_rm, user)
    w1u = W1[:, :F].T
    w1m = W1[:, F:2 * F].T
    w1g = W1[:, 2 * F:].T
    return _mlp(ue, me, ge, w1u, w1m, w1g,
                b1.reshape(1, 128), W2.T, b2.reshape(1, F),
                W3.T, b3.reshape(1, 1))


# speedup vs baseline: 1.0165x; 1.0165x over previous
import functools

import jax
import jax.numpy as jnp
from jax import lax
from jax.experimental import pallas as pl
from jax.experimental.pallas import tpu as pltpu
from jax.experimental.pallas import tpu_sc as plsc

B = 16384
F = 64
NC = 2
NS = 16
NW = NC * NS
BPW = B // NW
WIN = 48


@functools.lru_cache(maxsize=2)
def _make_gather(n_tables):
    mesh = plsc.VectorSubcoreMesh(core_axis_name="c", subcore_axis_name="s")

    @functools.partial(
        pl.kernel,
        mesh=mesh,
        out_type=[jax.ShapeDtypeStruct((B, F), jnp.float32)] * n_tables,
        scratch_types=[
            pltpu.VMEM((BPW + 16,), jnp.int32),
            pltpu.VMEM((BPW, F), jnp.float32),
            pltpu.SemaphoreType.DMA,
        ],
    )
    def _gather(*args):
        tables = args[:n_tables]
        idxs = args[n_tables:2 * n_tables]
        outs = args[2 * n_tables:3 * n_tables]
        iv, rows, sem = args[3 * n_tables:]
        wid = lax.axis_index("s") * NC + lax.axis_index("c")
        base = wid * BPW

        def one_table(table, idx, out):
            pltpu.sync_copy(idx.at[pl.ds(base, BPW)], iv.at[pl.ds(0, BPW)])

            def step(r, _):
                s = iv[pl.ds(r, 16)][0]
                pltpu.async_copy(
                    table.at[pl.ds(s, 1)], rows.at[pl.ds(r, 1)], sem)

                @pl.when(r >= WIN)
                def _():
                    pltpu.make_async_copy(
                        table.at[pl.ds(0, 1)], rows.at[pl.ds(0, 1)],
                        sem).wait()

                return 0

            lax.fori_loop(0, BPW, step, 0)
            for _ in range(WIN):
                pltpu.make_async_copy(
                    table.at[pl.ds(0, 1)], rows.at[pl.ds(0, 1)], sem).wait()
            pltpu.sync_copy(rows, out.at[pl.ds(base, BPW)])

        for t, i, o in zip(tables, idxs, outs):
            one_table(t, i, o)

    return _gather


def _transpose_body(t_ref, d_ref, o_ref):
    del d_ref
    eye = jnp.eye(F, dtype=jnp.float32)
    o_ref[...] = jax.lax.dot_general(
        t_ref[...], eye, (((0,), (0,)), ((), ())),
        preferred_element_type=jnp.float32)


def _transpose(tT, blk, dep=None):
    n = tT.shape[1]
    grid = (n + blk - 1) // blk
    if dep is None:
        dep = jnp.zeros((1, 1), jnp.float32)
    return pl.pallas_call(
        _transpose_body,
        grid=(grid,),
        in_specs=[
            pl.BlockSpec((F, blk), lambda i: (0, i)),
            pl.BlockSpec((1, 1), lambda i: (0, 0)),
        ],
        out_specs=pl.BlockSpec((blk, F), lambda i: (i, 0)),
        out_shape=jax.ShapeDtypeStruct((n, F), jnp.float32),
    )(tT, dep)


BT = 2048
GRID = B // BT


def _mlp_body(ue, me, ge, w1u, w1m, w1g, b1, w2, b2, w3, b3, out):
    x = (jnp.dot(ue[...], w1u[...], preferred_element_type=jnp.float32)
         + jnp.dot(me[...], w1m[...], preferred_element_type=jnp.float32)
         + jnp.dot(ge[...], w1g[...], preferred_element_type=jnp.float32)
         + b1[...])
    x = jnp.maximum(x, 0.0)
    x = jnp.maximum(
        jnp.dot(x, w2[...], preferred_element_type=jnp.float32) + b2[...], 0.0)
    out[...] = jnp.dot(x, w3[...], preferred_element_type=jnp.float32) + b3[...]


def _mlp(ue, me, ge, w1u, w1m, w1g, b1, w2, b2, w3, b3, *, interpret=False):
    full = lambda shape: pl.BlockSpec(shape, lambda i: (0, 0))
    return pl.pallas_call(
        _mlp_body,
        grid=(GRID,),
        in_specs=[
            pl.BlockSpec((BT, F), lambda i: (i, 0)),
            pl.BlockSpec((BT, F), lambda i: (i, 0)),
            pl.BlockSpec((BT, F), lambda i: (i, 0)),
            full((F, 128)),
            full((F, 128)),
            full((F, 128)),
            full((1, 128)),
            full((128, F)),
            full((1, F)),
            full((F, 1)),
            full((1, 1)),
        ],
        out_specs=pl.BlockSpec((BT, 1), lambda i: (i, 0)),
        out_shape=jax.ShapeDtypeStruct((B, 1), jnp.float32),
        interpret=interpret,
    )(ue, me, ge, w1u, w1m, w1g, b1, w2, b2, w3, b3)


def kernel(user, movie, genres, user_table, movie_table, genre_table,
           W1, b1, W2, b2, W3, b3):
    mt_rm = _transpose(movie_table.T, 16384)
    gt_rm = _transpose(genre_table.T, 1024)
    ut_rm = _transpose(user_table.T, 32768,
                       dep=mt_rm[:1, :1] + gt_rm[:1, :1])
    me, ge = _make_gather(2)(mt_rm, gt_rm, movie, genres)
    ue, = _make_gather(1)(ut_rm, user)
    w1u = W1[:, :F].T
    w1m = W1[:, F:2 * F].T
    w1g = W1[:, 2 * F:].T
    return _mlp(ue, me, ge, w1u, w1m, w1g,
                b1.reshape(1, 128), W2.T, b2.reshape(1, F),
                W3.T, b3.reshape(1, 1))
